# baseline (speedup 1.0000x reference)
import functools

import jax
import jax.numpy as jnp
from jax import lax
from jax.experimental import pallas as pl
from jax.experimental.pallas import tpu as pltpu
from jax.experimental.pallas import tpu_sc as plsc

B = 4
L = 4096
D = 1024
TABLE = 4096

NC = 2
NS = 16
NW = NC * NS
TOTAL = B * L
B_PER_W = TOTAL // NW
CH = 32
NCH = B_PER_W // CH


def _positions_body(ids_ref, pos_ref):
    ids = ids_ref[...]
    mask = (ids >= 48) & (ids <= 57)
    col = lax.broadcasted_iota(jnp.int32, ids.shape, 1)
    nd = jnp.where(mask, -1, col)
    sh = 1
    while sh < L:
        shifted = jnp.concatenate(
            [jnp.full((B, sh), -1, jnp.int32), nd[:, :-sh]], axis=1
        )
        nd = jnp.maximum(nd, shifted)
        sh *= 2
    pos = jnp.where(mask, col - nd, 0)
    pos_ref[...] = jnp.minimum(pos, TABLE - 1)


def _positions(input_ids):
    return pl.pallas_call(
        _positions_body,
        out_shape=jax.ShapeDtypeStruct((B, L), jnp.int32),
    )(input_ids)


@functools.cache
def _build_gather():
    return pl.kernel(
        _gather_body,
        out_type=jax.ShapeDtypeStruct((TOTAL, D), jnp.float32),
        mesh=plsc.VectorSubcoreMesh(core_axis_name="c", subcore_axis_name="s"),
        scratch_types=[
            pltpu.VMEM((B_PER_W,), jnp.int32),
            pltpu.VMEM((2, CH, D), jnp.float32),
            pltpu.SemaphoreType.DMA,
            pltpu.SemaphoreType.DMA,
            pltpu.SemaphoreType.DMA,
            pltpu.SemaphoreType.DMA,
        ],
    )


def _gather_body(idx_hbm, table_hbm, out_hbm, idx_v, rows_v, g0, g1, s0, s1):
    wid = lax.axis_index("s") * NC + lax.axis_index("c")
    base = wid * B_PER_W
    pltpu.sync_copy(idx_hbm.at[pl.ds(base, B_PER_W)], idx_v)
    gsem = [g0, g1]
    ssem = [s0, s1]
    gathers = [None, None]
    scatters = [None, None]
    for c in range(NCH):
        buf = c & 1
        if scatters[buf] is not None:
            scatters[buf].wait()
        gathers[buf] = pltpu.async_copy(
            table_hbm.at[idx_v.at[pl.ds(c * CH, CH)]], rows_v.at[buf], gsem[buf]
        )
        if c > 0:
            prev = (c - 1) & 1
            gathers[prev].wait()
            scatters[prev] = pltpu.async_copy(
                rows_v.at[prev], out_hbm.at[pl.ds(base + (c - 1) * CH, CH)],
                ssem[prev]
            )
    last = (NCH - 1) & 1
    gathers[last].wait()
    scatters[last] = pltpu.async_copy(
        rows_v.at[last], out_hbm.at[pl.ds(base + (NCH - 1) * CH, CH)], ssem[last]
    )
    scatters[last].wait()
    scatters[1 - last].wait()


def kernel(input_ids, embedding):
    pos = _positions(input_ids).reshape(TOTAL)
    out = _build_gather()(pos, embedding)
    return out.reshape(B, L, D)


# speedup vs baseline: 5.7603x; 5.7603x over previous
import functools

import jax
import jax.numpy as jnp
from jax import lax
from jax.experimental import pallas as pl
from jax.experimental.pallas import tpu as pltpu
from jax.experimental.pallas import tpu_sc as plsc

B = 4
L = 4096
D = 1024
TABLE = 4096

NC = 2
NS = 16
NW = NC * NS
TOTAL = B * L
B_PER_W = TOTAL // NW
FILL = 64
NFILL = B_PER_W // FILL
NGRP = B_PER_W // 16


def _positions_body(ids_ref, pos_ref):
    ids = ids_ref[...]
    mask = (ids >= 48) & (ids <= 57)
    col = lax.broadcasted_iota(jnp.int32, ids.shape, 1)
    nd = jnp.where(mask, -1, col)
    sh = 1
    while sh < L:
        shifted = jnp.concatenate(
            [jnp.full((B, sh), -1, jnp.int32), nd[:, :-sh]], axis=1
        )
        nd = jnp.maximum(nd, shifted)
        sh *= 2
    pos = jnp.where(mask, col - nd, 0)
    pos_ref[...] = jnp.minimum(pos, TABLE - 1)


def _positions(input_ids):
    return pl.pallas_call(
        _positions_body,
        out_shape=jax.ShapeDtypeStruct((B, L), jnp.int32),
    )(input_ids)


@functools.cache
def _build_fill_fix():
    return pl.kernel(
        _fill_fix_body,
        out_type=jax.ShapeDtypeStruct((TOTAL, D), jnp.float32),
        mesh=plsc.VectorSubcoreMesh(core_axis_name="c", subcore_axis_name="s"),
        scratch_types=[
            pltpu.VMEM((B_PER_W,), jnp.int32),
            pltpu.VMEM((FILL,), jnp.int32),
            pltpu.VMEM((FILL, D), jnp.float32),
            pltpu.VMEM((1, D), jnp.float32),
            pltpu.SemaphoreType.DMA,
            pltpu.SemaphoreType.DMA,
        ],
    )


def _fill_fix_body(idx_hbm, table_hbm, out_hbm, idx_v, zidx_v, buf_v, fix_v,
                   gsem, ssem):
    wid = lax.axis_index("s") * NC + lax.axis_index("c")
    base = wid * B_PER_W
    pltpu.sync_copy(idx_hbm.at[pl.ds(base, B_PER_W)], idx_v)
    for k in range(FILL // 16):
        zidx_v[pl.ds(k * 16, 16)] = jnp.zeros((16,), jnp.int32)
    pltpu.async_copy(table_hbm.at[zidx_v], buf_v, gsem).wait()
    fills = []
    for j in range(NFILL):
        fills.append(pltpu.async_copy(
            buf_v, out_hbm.at[pl.ds(base + j * FILL, FILL)], ssem
        ))
    for f in fills:
        f.wait()
    def group(g, _):
        vec = idx_v[pl.ds(g * 16, 16)]
        lane = [vec[r] for r in range(16)]
        any_nz = lane[0]
        for r in range(1, 16):
            any_nz = any_nz | lane[r]

        @pl.when(any_nz > 0)
        def _():
            for r in range(16):
                @pl.when(lane[r] > 0)
                def _(r=r):
                    pltpu.sync_copy(table_hbm.at[pl.ds(lane[r], 1)], fix_v)
                    pltpu.sync_copy(
                        fix_v, out_hbm.at[pl.ds(base + g * 16 + r, 1)]
                    )
        return 0

    lax.fori_loop(0, NGRP, group, 0)


def kernel(input_ids, embedding):
    pos = _positions(input_ids).reshape(TOTAL)
    out = _build_fill_fix()(pos, embedding)
    return out.reshape(B, L, D)


# speedup vs baseline: 9.8203x; 1.7048x over previous
import functools

import jax
import jax.numpy as jnp
from jax import lax
from jax.experimental import pallas as pl
from jax.experimental.pallas import tpu as pltpu
from jax.experimental.pallas import tpu_sc as plsc

B = 4
L = 4096
D = 1024
TABLE = 4096

NC = 2
NS = 16
NW = NC * NS
TOTAL = B * L
B_PER_W = TOTAL // NW
FILL = 16
NFILL = B_PER_W // FILL
NGRP = B_PER_W // 16


def _positions_body(ids_ref, pos_ref):
    ids = ids_ref[...]
    mask = (ids >= 48) & (ids <= 57)
    col = lax.broadcasted_iota(jnp.int32, ids.shape, 1)
    nd = jnp.where(mask, -1, col)
    sh = 1
    while sh < L:
        shifted = jnp.concatenate(
            [jnp.full((B, sh), -1, jnp.int32), nd[:, :-sh]], axis=1
        )
        nd = jnp.maximum(nd, shifted)
        sh *= 2
    pos = jnp.where(mask, col - nd, 0)
    pos_ref[...] = jnp.minimum(pos, TABLE - 1)


def _positions(input_ids):
    return pl.pallas_call(
        _positions_body,
        out_shape=jax.ShapeDtypeStruct((B, L), jnp.int32),
    )(input_ids)


@functools.cache
def _build_fill_fix():
    return pl.kernel(
        _fill_fix_body,
        out_type=jax.ShapeDtypeStruct((TOTAL, D), jnp.float32),
        mesh=plsc.VectorSubcoreMesh(core_axis_name="c", subcore_axis_name="s"),
        scratch_types=[
            pltpu.VMEM((B_PER_W,), jnp.int32),
            pltpu.VMEM((FILL,), jnp.int32),
            pltpu.VMEM((FILL, D), jnp.float32),
            pltpu.VMEM((1, D), jnp.float32),
            pltpu.SemaphoreType.DMA,
            pltpu.SemaphoreType.DMA,
        ],
    )


def _fill_fix_body(idx_hbm, table_hbm, out_hbm, idx_v, zidx_v, buf_v, fix_v,
                   gsem, ssem):
    wid = lax.axis_index("s") * NC + lax.axis_index("c")
    base = wid * B_PER_W
    pltpu.sync_copy(idx_hbm.at[pl.ds(base, B_PER_W)], idx_v)
    for k in range(FILL // 16):
        zidx_v[pl.ds(k * 16, 16)] = jnp.zeros((16,), jnp.int32)
    pltpu.async_copy(table_hbm.at[zidx_v], buf_v, gsem).wait()
    fills = []
    for j in range(NFILL):
        fills.append(pltpu.async_copy(
            buf_v, out_hbm.at[pl.ds(base + j * FILL, FILL)], ssem
        ))
    for f in fills:
        f.wait()
    def group(g, _):
        vec = idx_v[pl.ds(g * 16, 16)]
        lane = [vec[r] for r in range(16)]
        any_nz = lane[0]
        for r in range(1, 16):
            any_nz = any_nz | lane[r]

        @pl.when(any_nz > 0)
        def _():
            for r in range(16):
                @pl.when(lane[r] > 0)
                def _(r=r):
                    pltpu.sync_copy(table_hbm.at[pl.ds(lane[r], 1)], fix_v)
                    pltpu.sync_copy(
                        fix_v, out_hbm.at[pl.ds(base + g * 16 + r, 1)]
                    )
        return 0

    lax.fori_loop(0, NGRP, group, 0)


def kernel(input_ids, embedding):
    pos = _positions(input_ids).reshape(TOTAL)
    out = _build_fill_fix()(pos, embedding)
    return out.reshape(B, L, D)


# speedup vs baseline: 11.2868x; 1.1493x over previous
import functools

import jax
import jax.numpy as jnp
from jax import lax
from jax.experimental import pallas as pl
from jax.experimental.pallas import tpu as pltpu
from jax.experimental.pallas import tpu_sc as plsc

B = 4
L = 4096
D = 1024
TABLE = 4096

NC = 2
NS = 16
NW = NC * NS
TOTAL = B * L
B_PER_W = TOTAL // NW
FILL = 8
NFILL = B_PER_W // FILL
NGRP = B_PER_W // 16


def _positions_body(ids_ref, pos_ref):
    ids = ids_ref[...]
    mask = (ids >= 48) & (ids <= 57)
    col = lax.broadcasted_iota(jnp.int32, ids.shape, 1)
    nd = jnp.where(mask, -1, col)
    sh = 1
    while sh < L:
        shifted = jnp.concatenate(
            [jnp.full((B, sh), -1, jnp.int32), nd[:, :-sh]], axis=1
        )
        nd = jnp.maximum(nd, shifted)
        sh *= 2
    pos = jnp.where(mask, col - nd, 0)
    pos_ref[...] = jnp.minimum(pos, TABLE - 1)


def _positions(input_ids):
    return pl.pallas_call(
        _positions_body,
        out_shape=jax.ShapeDtypeStruct((B, L), jnp.int32),
    )(input_ids)


@functools.cache
def _build_fill_fix():
    return pl.kernel(
        _fill_fix_body,
        out_type=jax.ShapeDtypeStruct((TOTAL, D), jnp.float32),
        mesh=plsc.VectorSubcoreMesh(core_axis_name="c", subcore_axis_name="s"),
        scratch_types=[
            pltpu.VMEM((B_PER_W,), jnp.int32),
            pltpu.VMEM((max(FILL, 16),), jnp.int32),
            pltpu.VMEM((FILL, D), jnp.float32),
            pltpu.VMEM((1, D), jnp.float32),
            pltpu.SemaphoreType.DMA,
            pltpu.SemaphoreType.DMA,
        ],
    )


def _fill_fix_body(idx_hbm, table_hbm, out_hbm, idx_v, zidx_v, buf_v, fix_v,
                   gsem, ssem):
    wid = lax.axis_index("s") * NC + lax.axis_index("c")
    base = wid * B_PER_W
    pltpu.sync_copy(idx_hbm.at[pl.ds(base, B_PER_W)], idx_v)
    for k in range(max(FILL, 16) // 16):
        zidx_v[pl.ds(k * 16, 16)] = jnp.zeros((16,), jnp.int32)
    pltpu.async_copy(
        table_hbm.at[zidx_v.at[pl.ds(0, FILL)]], buf_v, gsem
    ).wait()
    fills = []
    for j in range(NFILL):
        fills.append(pltpu.async_copy(
            buf_v, out_hbm.at[pl.ds(base + j * FILL, FILL)], ssem
        ))
    for f in fills:
        f.wait()
    def group(g, _):
        vec = idx_v[pl.ds(g * 16, 16)]
        lane = [vec[r] for r in range(16)]
        any_nz = lane[0]
        for r in range(1, 16):
            any_nz = any_nz | lane[r]

        @pl.when(any_nz > 0)
        def _():
            for r in range(16):
                @pl.when(lane[r] > 0)
                def _(r=r):
                    pltpu.sync_copy(table_hbm.at[pl.ds(lane[r], 1)], fix_v)
                    pltpu.sync_copy(
                        fix_v, out_hbm.at[pl.ds(base + g * 16 + r, 1)]
                    )
        return 0

    lax.fori_loop(0, NGRP, group, 0)


def kernel(input_ids, embedding):
    pos = _positions(input_ids).reshape(TOTAL)
    out = _build_fill_fix()(pos, embedding)
    return out.reshape(B, L, D)


# speedup vs baseline: 11.9516x; 1.0589x over previous
import functools

import jax
import jax.numpy as jnp
from jax import lax
from jax.experimental import pallas as pl
from jax.experimental.pallas import tpu as pltpu
from jax.experimental.pallas import tpu_sc as plsc

B = 4
L = 4096
D = 1024
TABLE = 4096

NC = 2
NS = 16
NW = NC * NS
TOTAL = B * L
B_PER_W = TOTAL // NW
FILL = 4
NFILL = B_PER_W // FILL
NGRP = B_PER_W // 16


def _positions_body(ids_ref, pos_ref):
    ids = ids_ref[...]
    mask = (ids >= 48) & (ids <= 57)
    col = lax.broadcasted_iota(jnp.int32, ids.shape, 1)
    nd = jnp.where(mask, -1, col)
    sh = 1
    while sh < L:
        shifted = jnp.concatenate(
            [jnp.full((B, sh), -1, jnp.int32), nd[:, :-sh]], axis=1
        )
        nd = jnp.maximum(nd, shifted)
        sh *= 2
    pos = jnp.where(mask, col - nd, 0)
    pos_ref[...] = jnp.minimum(pos, TABLE - 1)


def _positions(input_ids):
    return pl.pallas_call(
        _positions_body,
        out_shape=jax.ShapeDtypeStruct((B, L), jnp.int32),
    )(input_ids)


@functools.cache
def _build_fill_fix():
    return pl.kernel(
        _fill_fix_body,
        out_type=jax.ShapeDtypeStruct((TOTAL, D), jnp.float32),
        mesh=plsc.VectorSubcoreMesh(core_axis_name="c", subcore_axis_name="s"),
        scratch_types=[
            pltpu.VMEM((B_PER_W,), jnp.int32),
            pltpu.VMEM((max(FILL, 16),), jnp.int32),
            pltpu.VMEM((FILL, D), jnp.float32),
            pltpu.VMEM((1, D), jnp.float32),
            pltpu.SemaphoreType.DMA,
            pltpu.SemaphoreType.DMA,
        ],
    )


def _fill_fix_body(idx_hbm, table_hbm, out_hbm, idx_v, zidx_v, buf_v, fix_v,
                   gsem, ssem):
    wid = lax.axis_index("s") * NC + lax.axis_index("c")
    base = wid * B_PER_W
    pltpu.sync_copy(idx_hbm.at[pl.ds(base, B_PER_W)], idx_v)
    for k in range(max(FILL, 16) // 16):
        zidx_v[pl.ds(k * 16, 16)] = jnp.zeros((16,), jnp.int32)
    pltpu.async_copy(
        table_hbm.at[zidx_v.at[pl.ds(0, FILL)]], buf_v, gsem
    ).wait()
    fills = []
    for j in range(NFILL):
        fills.append(pltpu.async_copy(
            buf_v, out_hbm.at[pl.ds(base + j * FILL, FILL)], ssem
        ))
    for f in fills:
        f.wait()
    def group(g, _):
        vec = idx_v[pl.ds(g * 16, 16)]
        lane = [vec[r] for r in range(16)]
        any_nz = lane[0]
        for r in range(1, 16):
            any_nz = any_nz | lane[r]

        @pl.when(any_nz > 0)
        def _():
            for r in range(16):
                @pl.when(lane[r] > 0)
                def _(r=r):
                    pltpu.sync_copy(table_hbm.at[pl.ds(lane[r], 1)], fix_v)
                    pltpu.sync_copy(
                        fix_v, out_hbm.at[pl.ds(base + g * 16 + r, 1)]
                    )
        return 0

    lax.fori_loop(0, NGRP, group, 0)


def kernel(input_ids, embedding):
    pos = _positions(input_ids).reshape(TOTAL)
    out = _build_fill_fix()(pos, embedding)
    return out.reshape(B, L, D)


# speedup vs baseline: 11.9954x; 1.0037x over previous
import functools

import jax
import jax.numpy as jnp
from jax import lax
from jax.experimental import pallas as pl
from jax.experimental.pallas import tpu as pltpu
from jax.experimental.pallas import tpu_sc as plsc

B = 4
L = 4096
D = 1024
TABLE = 4096

NC = 2
NS = 16
NW = NC * NS
TOTAL = B * L
B_PER_W = TOTAL // NW
FILL = 4
NFILL = B_PER_W // FILL
NGRP = B_PER_W // 16


def _positions_body(ids_ref, pos_ref):
    ids = ids_ref[...]
    mask = (ids >= 48) & (ids <= 57)
    col = lax.broadcasted_iota(jnp.int32, ids.shape, 1)
    nd = jnp.where(mask, -1, col)
    sh = 1
    while sh < L:
        shifted = jnp.concatenate(
            [jnp.full((B, sh), -1, jnp.int32), nd[:, :-sh]], axis=1
        )
        nd = jnp.maximum(nd, shifted)
        sh *= 2
    pos = jnp.where(mask, col - nd, 0)
    pos_ref[...] = jnp.minimum(pos, TABLE - 1)


def _positions(input_ids):
    return pl.pallas_call(
        _positions_body,
        out_shape=jax.ShapeDtypeStruct((B, L), jnp.int32),
    )(input_ids)


@functools.cache
def _build_fill_fix():
    return pl.kernel(
        _fill_fix_body,
        out_type=jax.ShapeDtypeStruct((TOTAL, D), jnp.float32),
        mesh=plsc.VectorSubcoreMesh(core_axis_name="c", subcore_axis_name="s"),
        scratch_types=[
            pltpu.VMEM((1, B_PER_W), jnp.int32),
            pltpu.VMEM((max(FILL, 16),), jnp.int32),
            pltpu.VMEM((FILL, D), jnp.float32),
            pltpu.VMEM((1, D), jnp.float32),
            pltpu.SemaphoreType.DMA,
            pltpu.SemaphoreType.DMA,
        ],
    )


def _fill_fix_body(idx_hbm, table_hbm, out_hbm, idx_v, zidx_v, buf_v, fix_v,
                   gsem, ssem):
    wid = lax.axis_index("s") * NC + lax.axis_index("c")
    base = wid * B_PER_W
    b = wid // (L // B_PER_W)
    j0 = (wid % (L // B_PER_W)) * B_PER_W
    pltpu.sync_copy(idx_hbm.at[pl.ds(b, 1), pl.ds(j0, B_PER_W)], idx_v)
    for k in range(max(FILL, 16) // 16):
        zidx_v[pl.ds(k * 16, 16)] = jnp.zeros((16,), jnp.int32)
    pltpu.async_copy(
        table_hbm.at[zidx_v.at[pl.ds(0, FILL)]], buf_v, gsem
    ).wait()
    fills = []
    for j in range(NFILL):
        fills.append(pltpu.async_copy(
            buf_v, out_hbm.at[pl.ds(base + j * FILL, FILL)], ssem
        ))
    for f in fills:
        f.wait()
    def group(g, _):
        vec = idx_v[0, pl.ds(g * 16, 16)]
        lane = [vec[r] for r in range(16)]
        any_nz = lane[0]
        for r in range(1, 16):
            any_nz = any_nz | lane[r]

        @pl.when(any_nz > 0)
        def _():
            for r in range(16):
                @pl.when(lane[r] > 0)
                def _(r=r):
                    pltpu.sync_copy(table_hbm.at[pl.ds(lane[r], 1)], fix_v)
                    pltpu.sync_copy(
                        fix_v, out_hbm.at[pl.ds(base + g * 16 + r, 1)]
                    )
        return 0

    lax.fori_loop(0, NGRP, group, 0)


def kernel(input_ids, embedding):
    pos = _positions(input_ids)
    out = _build_fill_fix()(pos, embedding)
    return out.reshape(B, L, D)
